# baseline (speedup 1.0000x reference)
import functools

import jax
import jax.numpy as jnp
from jax import lax
from jax.experimental import pallas as pl
from jax.experimental.pallas import tpu as pltpu
from jax.experimental.pallas import tpu_sc as plsc

VOCAB = 100000
EMBED = 200
EMBED_P = 256
HIDDEN = 64
CLASSES = 100
BATCH = 4096
SEQ = 50

LANES = 16
NCORES = 2
NSUB = 16
NWORKERS = NCORES * NSUB
ROWS_PER_W = BATCH // NWORKERS
NSEG = 2 * SEQ
NCH_LO = 128 // LANES
NCH_HI = (EMBED - 128) // LANES
HI_TAIL = EMBED - 128 - LANES


NBUF = 4


def _pool_body(seg_hbm, table_hbm, out_hbm, idx_v, bufs, out_v, sems):
    cid = lax.axis_index("c")
    sid = lax.axis_index("s")
    wid = sid * NCORES + cid
    base = wid * ROWS_PER_W

    pltpu.sync_copy(seg_hbm.at[pl.ds(base, ROWS_PER_W)], idx_v)

    for b in range(NBUF - 1):
        pltpu.async_copy(table_hbm.at[idx_v.at[b]], bufs[b], sems[b])

    inv = jnp.full((LANES,), 1.0 / SEQ, dtype=jnp.float32)

    def accumulate(buf, i):
        lo = [buf[0, pl.ds(k * LANES, LANES)] for k in range(NCH_LO)]
        hi = [buf[SEQ, pl.ds(k * LANES, LANES)] for k in range(NCH_HI)]
        ht = buf[SEQ, pl.ds(HI_TAIL, LANES)]
        for j in range(1, SEQ):
            lo = [lo[k] + buf[j, pl.ds(k * LANES, LANES)]
                  for k in range(NCH_LO)]
            hi = [hi[k] + buf[SEQ + j, pl.ds(k * LANES, LANES)]
                  for k in range(NCH_HI)]
            ht = ht + buf[SEQ + j, pl.ds(HI_TAIL, LANES)]
        for k in range(NCH_LO):
            out_v[2 * i, pl.ds(k * LANES, LANES)] = lo[k] * inv
        for k in range(NCH_HI):
            out_v[2 * i + 1, pl.ds(k * LANES, LANES)] = hi[k] * inv
        out_v[2 * i + 1, pl.ds(HI_TAIL, LANES)] = ht * inv

    def loop_body(j, carry):
        for b in range(NBUF):
            i = j + b
            nxt_row = jnp.minimum(i + NBUF - 1, ROWS_PER_W - 1)
            nb = (b + NBUF - 1) % NBUF
            pltpu.async_copy(table_hbm.at[idx_v.at[nxt_row]], bufs[nb],
                             sems[nb])
            pltpu.make_async_copy(table_hbm.at[idx_v.at[i]], bufs[b],
                                  sems[b]).wait()
            accumulate(bufs[b], i)
        return carry

    lax.fori_loop(0, ROWS_PER_W // NBUF,
                  lambda t, c: loop_body(t * NBUF, c), 0, unroll=False)

    for b in range(NBUF - 1):
        pltpu.make_async_copy(table_hbm.at[idx_v.at[0]], bufs[b],
                              sems[b]).wait()

    pltpu.sync_copy(out_v, out_hbm.at[pl.ds(2 * base, 2 * ROWS_PER_W)])


@functools.partial(
    pl.kernel,
    out_type=jax.ShapeDtypeStruct((2 * BATCH, 128), jnp.float32),
    mesh=plsc.VectorSubcoreMesh(core_axis_name="c", subcore_axis_name="s"),
    scratch_types=[
        pltpu.VMEM((ROWS_PER_W, NSEG), jnp.int32),
        [pltpu.VMEM((NSEG, 128), jnp.float32)] * NBUF,
        pltpu.VMEM((2 * ROWS_PER_W, 128), jnp.float32),
        [pltpu.SemaphoreType.DMA] * NBUF,
    ],
)
def _pool(seg_hbm, table_hbm, out_hbm, idx_v, bufs, out_v, sems):
    _pool_body(seg_hbm, table_hbm, out_hbm, idx_v, bufs, out_v, sems)


RFMT_BLK = 2000


def _reformat_body(e_ref, o_ref):
    x = e_ref[...]
    y = jnp.pad(x, ((0, 0), (0, EMBED_P - EMBED)))
    o_ref[...] = y.reshape(2 * RFMT_BLK, 128)


def _reformat(embed):
    return pl.pallas_call(
        _reformat_body,
        grid=(VOCAB // RFMT_BLK,),
        in_specs=[pl.BlockSpec((RFMT_BLK, EMBED), lambda i: (i, 0))],
        out_specs=pl.BlockSpec((2 * RFMT_BLK, 128), lambda i: (i, 0)),
        out_shape=jax.ShapeDtypeStruct((2 * VOCAB, 128), jnp.float32),
    )(embed)


MLP_BLK = 1024


def _mlp_body(p_ref, w1_ref, b1_ref, w2_ref, b2_ref, o_ref):
    h = jnp.dot(p_ref[...], w1_ref[...],
                preferred_element_type=jnp.float32) + b1_ref[...]
    h = jnp.maximum(h, 0.0)
    z = jnp.dot(h, w2_ref[...],
                preferred_element_type=jnp.float32) + b2_ref[...]
    m = jnp.max(z, axis=1, keepdims=True)
    lse = jnp.log(jnp.sum(jnp.exp(z - m), axis=1, keepdims=True)) + m
    o_ref[...] = z - lse


def _mlp(pooled, W1, b1, W2, b2):
    return pl.pallas_call(
        _mlp_body,
        grid=(BATCH // MLP_BLK,),
        in_specs=[
            pl.BlockSpec((MLP_BLK, EMBED), lambda i: (i, 0)),
            pl.BlockSpec((EMBED, HIDDEN), lambda i: (0, 0)),
            pl.BlockSpec((1, HIDDEN), lambda i: (0, 0)),
            pl.BlockSpec((HIDDEN, CLASSES), lambda i: (0, 0)),
            pl.BlockSpec((1, CLASSES), lambda i: (0, 0)),
        ],
        out_specs=pl.BlockSpec((MLP_BLK, CLASSES), lambda i: (i, 0)),
        out_shape=jax.ShapeDtypeStruct((BATCH, CLASSES), jnp.float32),
    )(pooled, W1, b1.reshape(1, HIDDEN), W2, b2.reshape(1, CLASSES))


def kernel(x, embed, W1, b1, W2, b2):
    xi = x.astype(jnp.int32)
    seg = jnp.concatenate([2 * xi, 2 * xi + 1], axis=1)
    table = _reformat(embed)
    pooled2 = _pool(seg, table)
    pooled = pooled2.reshape(BATCH, EMBED_P)[:, :EMBED]
    return _mlp(pooled, W1, b1, W2, b2)


# speedup vs baseline: 2.0288x; 2.0288x over previous
import functools

import jax
import jax.numpy as jnp
from jax import lax
from jax.experimental import pallas as pl
from jax.experimental.pallas import tpu as pltpu
from jax.experimental.pallas import tpu_sc as plsc

VOCAB = 100000
EMBED = 200
EMBED_P = 256
HIDDEN = 64
CLASSES = 100
BATCH = 4096
SEQ = 50

LANES = 16
NCORES = 2
NSUB = 16
NWORKERS = NCORES * NSUB
ROWS_PER_W = BATCH // NWORKERS
NBLK = 128 // LANES
NBLK_HI = 5


NBUF = 2


def _pool_body(x_hbm, table_hbm, out_hbm, idx_v, bufs, out_v, sems):
    cid = lax.axis_index("c")
    sid = lax.axis_index("s")
    wid = sid * NCORES + cid
    base = wid * ROWS_PER_W

    pltpu.sync_copy(x_hbm.at[pl.ds(base, ROWS_PER_W)], idx_v)

    for b in range(NBUF - 1):
        pltpu.async_copy(table_hbm.at[idx_v.at[b]], bufs[b], sems[b])

    inv = jnp.full((LANES,), 1.0 / SEQ, dtype=jnp.float32)

    def block(buf, j, k):
        v = plsc.bitcast(buf[j, pl.ds(k * LANES, LANES)],
                         jnp.bfloat16)
        return plsc.unpack(v, format=plsc.PackFormat.INTERLEAVED)

    def accumulate(buf, i):
        acc = [block(buf, 0, k) for k in range(NBLK)]
        for j in range(1, SEQ):
            for k in range(NBLK):
                a, b = block(buf, j, k)
                bk = acc[k][1] + b if k < NBLK_HI else acc[k][1]
                acc[k] = (acc[k][0] + a, bk)
        for k in range(NBLK):
            out_v[2 * i, pl.ds(k * LANES, LANES)] = acc[k][0] * inv
        for k in range(NBLK_HI):
            out_v[2 * i + 1, pl.ds(k * LANES, LANES)] = acc[k][1] * inv

    def loop_body(j, carry):
        for b in range(NBUF):
            i = j + b
            nxt_row = jnp.minimum(i + NBUF - 1, ROWS_PER_W - 1)
            nb = (b + NBUF - 1) % NBUF
            pltpu.async_copy(table_hbm.at[idx_v.at[nxt_row]], bufs[nb],
                             sems[nb])
            pltpu.make_async_copy(table_hbm.at[idx_v.at[i]], bufs[b],
                                  sems[b]).wait()
            accumulate(bufs[b], i)
        return carry

    lax.fori_loop(0, ROWS_PER_W // NBUF,
                  lambda t, c: loop_body(t * NBUF, c), 0, unroll=False)

    for b in range(NBUF - 1):
        pltpu.make_async_copy(table_hbm.at[idx_v.at[0]], bufs[b],
                              sems[b]).wait()

    pltpu.sync_copy(out_v, out_hbm.at[pl.ds(2 * base, 2 * ROWS_PER_W)])


@functools.partial(
    pl.kernel,
    out_type=jax.ShapeDtypeStruct((2 * BATCH, 128), jnp.float32),
    mesh=plsc.VectorSubcoreMesh(core_axis_name="c", subcore_axis_name="s"),
    compiler_params=pltpu.CompilerParams(needs_layout_passes=False),
    scratch_types=[
        pltpu.VMEM((ROWS_PER_W, SEQ), jnp.int32),
        [pltpu.VMEM((SEQ, 128), jnp.int32)] * NBUF,
        pltpu.VMEM((2 * ROWS_PER_W, 128), jnp.float32),
        [pltpu.SemaphoreType.DMA] * NBUF,
    ],
)
def _pool(x_hbm, table_hbm, out_hbm, idx_v, bufs, out_v, sems):
    _pool_body(x_hbm, table_hbm, out_hbm, idx_v, bufs, out_v, sems)


RFMT_BLK = 2000


def _reformat_body(e_ref, o_ref):
    x = e_ref[...]
    y = jnp.pad(x, ((0, 0), (0, EMBED_P - EMBED)))
    y = y.astype(jnp.bfloat16)
    lo = jax.lax.bitcast_convert_type(y[:, :128], jnp.uint16)
    hi = jax.lax.bitcast_convert_type(y[:, 128:], jnp.uint16)
    o_ref[...] = (lo.astype(jnp.int32)
                  | (hi.astype(jnp.int32) << 16))


def _reformat(embed):
    return pl.pallas_call(
        _reformat_body,
        grid=(VOCAB // RFMT_BLK,),
        in_specs=[pl.BlockSpec((RFMT_BLK, EMBED), lambda i: (i, 0))],
        out_specs=pl.BlockSpec((RFMT_BLK, 128), lambda i: (i, 0)),
        out_shape=jax.ShapeDtypeStruct((VOCAB, 128), jnp.int32),
    )(embed)


MLP_BLK = 1024


def _mlp_body(p_ref, w1_ref, b1_ref, w2_ref, b2_ref, o_ref):
    h = jnp.dot(p_ref[...], w1_ref[...],
                preferred_element_type=jnp.float32) + b1_ref[...]
    h = jnp.maximum(h, 0.0)
    z = jnp.dot(h, w2_ref[...],
                preferred_element_type=jnp.float32) + b2_ref[...]
    m = jnp.max(z, axis=1, keepdims=True)
    lse = jnp.log(jnp.sum(jnp.exp(z - m), axis=1, keepdims=True)) + m
    o_ref[...] = z - lse


def _mlp(pooled, W1, b1, W2, b2):
    return pl.pallas_call(
        _mlp_body,
        grid=(BATCH // MLP_BLK,),
        in_specs=[
            pl.BlockSpec((MLP_BLK, EMBED), lambda i: (i, 0)),
            pl.BlockSpec((EMBED, HIDDEN), lambda i: (0, 0)),
            pl.BlockSpec((1, HIDDEN), lambda i: (0, 0)),
            pl.BlockSpec((HIDDEN, CLASSES), lambda i: (0, 0)),
            pl.BlockSpec((1, CLASSES), lambda i: (0, 0)),
        ],
        out_specs=pl.BlockSpec((MLP_BLK, CLASSES), lambda i: (i, 0)),
        out_shape=jax.ShapeDtypeStruct((BATCH, CLASSES), jnp.float32),
    )(pooled, W1, b1.reshape(1, HIDDEN), W2, b2.reshape(1, CLASSES))


def kernel(x, embed, W1, b1, W2, b2):
    xi = x.astype(jnp.int32)
    table = _reformat(embed)
    pooled2 = _pool(xi, table)
    pooled = pooled2.reshape(BATCH, EMBED_P)[:, :EMBED]
    return _mlp(pooled, W1, b1, W2, b2)


# speedup vs baseline: 2.1344x; 1.0520x over previous
import functools

import jax
import jax.numpy as jnp
from jax import lax
from jax.experimental import pallas as pl
from jax.experimental.pallas import tpu as pltpu
from jax.experimental.pallas import tpu_sc as plsc

VOCAB = 100000
EMBED = 200
EMBED_P = 256
HIDDEN = 64
CLASSES = 100
BATCH = 4096
SEQ = 50

LANES = 16
NCORES = 2
NSUB = 16
NWORKERS = NCORES * NSUB
ROWS_PER_W = BATCH // NWORKERS
NBLK = 128 // LANES
NBLK_HI = 5


NBUF = 2


def _pool_body(x_hbm, table_hbm, out_hbm, idx_v, bufs, out_v, sems):
    cid = lax.axis_index("c")
    sid = lax.axis_index("s")
    wid = sid * NCORES + cid
    base = wid * ROWS_PER_W

    pltpu.sync_copy(x_hbm.at[pl.ds(base, ROWS_PER_W)], idx_v)

    for b in range(NBUF - 1):
        pltpu.async_copy(table_hbm.at[idx_v.at[b]], bufs[b], sems[b])

    inv = jnp.full((LANES,), 1.0 / SEQ, dtype=jnp.float32)

    def block(buf, j, k):
        v = plsc.bitcast(buf[j, pl.ds(k * LANES, LANES)],
                         jnp.bfloat16)
        return plsc.unpack(v, format=plsc.PackFormat.INTERLEAVED)

    def accumulate(buf, i):
        acc = [block(buf, 0, k) for k in range(NBLK)]
        for j in range(1, SEQ):
            for k in range(NBLK):
                a, b = block(buf, j, k)
                bk = acc[k][1] + b if k < NBLK_HI else acc[k][1]
                acc[k] = (acc[k][0] + a, bk)
        for k in range(NBLK):
            out_v[2 * i, pl.ds(k * LANES, LANES)] = acc[k][0] * inv
        for k in range(NBLK_HI):
            out_v[2 * i + 1, pl.ds(k * LANES, LANES)] = acc[k][1] * inv

    def loop_body(j, carry):
        for b in range(NBUF):
            i = j + b
            nxt_row = jnp.minimum(i + NBUF - 1, ROWS_PER_W - 1)
            nb = (b + NBUF - 1) % NBUF
            pltpu.async_copy(table_hbm.at[idx_v.at[nxt_row]], bufs[nb],
                             sems[nb])
            pltpu.make_async_copy(table_hbm.at[idx_v.at[i]], bufs[b],
                                  sems[b]).wait()
            accumulate(bufs[b], i)
        return carry

    lax.fori_loop(0, ROWS_PER_W // NBUF,
                  lambda t, c: loop_body(t * NBUF, c), 0, unroll=False)

    for b in range(NBUF - 1):
        pltpu.make_async_copy(table_hbm.at[idx_v.at[0]], bufs[b],
                              sems[b]).wait()

    pltpu.sync_copy(out_v, out_hbm.at[pl.ds(2 * base, 2 * ROWS_PER_W)])


@functools.partial(
    pl.kernel,
    out_type=jax.ShapeDtypeStruct((2 * BATCH, 128), jnp.float32),
    mesh=plsc.VectorSubcoreMesh(core_axis_name="c", subcore_axis_name="s"),
    compiler_params=pltpu.CompilerParams(needs_layout_passes=False),
    scratch_types=[
        pltpu.VMEM((ROWS_PER_W, SEQ), jnp.int32),
        [pltpu.VMEM((SEQ, 128), jnp.int32)] * NBUF,
        pltpu.VMEM((2 * ROWS_PER_W, 128), jnp.float32),
        [pltpu.SemaphoreType.DMA] * NBUF,
    ],
)
def _pool(x_hbm, table_hbm, out_hbm, idx_v, bufs, out_v, sems):
    _pool_body(x_hbm, table_hbm, out_hbm, idx_v, bufs, out_v, sems)


RFMT_BLK = 5000


def _reformat_body(e_ref, o_ref):
    x = e_ref[...]
    y = jnp.pad(x, ((0, 0), (0, EMBED_P - EMBED)))
    y = y.astype(jnp.bfloat16)
    lo = jax.lax.bitcast_convert_type(y[:, :128], jnp.uint16)
    hi = jax.lax.bitcast_convert_type(y[:, 128:], jnp.uint16)
    o_ref[...] = (lo.astype(jnp.int32)
                  | (hi.astype(jnp.int32) << 16))


def _reformat(embed):
    return pl.pallas_call(
        _reformat_body,
        grid=(VOCAB // RFMT_BLK,),
        in_specs=[pl.BlockSpec((RFMT_BLK, EMBED), lambda i: (i, 0))],
        out_specs=pl.BlockSpec((RFMT_BLK, 128), lambda i: (i, 0)),
        out_shape=jax.ShapeDtypeStruct((VOCAB, 128), jnp.int32),
    )(embed)


MLP_BLK = 1024


def _mlp_body(p_ref, w1_ref, b1_ref, w2_ref, b2_ref, o_ref):
    h = jnp.dot(p_ref[...], w1_ref[...],
                preferred_element_type=jnp.float32) + b1_ref[...]
    h = jnp.maximum(h, 0.0)
    z = jnp.dot(h, w2_ref[...],
                preferred_element_type=jnp.float32) + b2_ref[...]
    m = jnp.max(z, axis=1, keepdims=True)
    lse = jnp.log(jnp.sum(jnp.exp(z - m), axis=1, keepdims=True)) + m
    o_ref[...] = z - lse


def _mlp(pooled, W1, b1, W2, b2):
    return pl.pallas_call(
        _mlp_body,
        grid=(BATCH // MLP_BLK,),
        in_specs=[
            pl.BlockSpec((MLP_BLK, EMBED), lambda i: (i, 0)),
            pl.BlockSpec((EMBED, HIDDEN), lambda i: (0, 0)),
            pl.BlockSpec((1, HIDDEN), lambda i: (0, 0)),
            pl.BlockSpec((HIDDEN, CLASSES), lambda i: (0, 0)),
            pl.BlockSpec((1, CLASSES), lambda i: (0, 0)),
        ],
        out_specs=pl.BlockSpec((MLP_BLK, CLASSES), lambda i: (i, 0)),
        out_shape=jax.ShapeDtypeStruct((BATCH, CLASSES), jnp.float32),
    )(pooled, W1, b1.reshape(1, HIDDEN), W2, b2.reshape(1, CLASSES))


def kernel(x, embed, W1, b1, W2, b2):
    xi = x.astype(jnp.int32)
    table = _reformat(embed)
    pooled2 = _pool(xi, table)
    pooled = pooled2.reshape(BATCH, EMBED_P)[:, :EMBED]
    return _mlp(pooled, W1, b1, W2, b2)


# speedup vs baseline: 2.1355x; 1.0005x over previous
import functools

import jax
import jax.numpy as jnp
from jax import lax
from jax.experimental import pallas as pl
from jax.experimental.pallas import tpu as pltpu
from jax.experimental.pallas import tpu_sc as plsc

VOCAB = 100000
EMBED = 200
EMBED_P = 256
HIDDEN = 64
CLASSES = 100
BATCH = 4096
SEQ = 50

LANES = 16
NCORES = 2
NSUB = 16
NWORKERS = NCORES * NSUB
ROWS_PER_W = BATCH // NWORKERS
NBLK = 128 // LANES
NBLK_HI = 5


NBUF = 2


def _pool_body(x_hbm, table_hbm, out_hbm, idx_v, bufs, out_v, sems):
    cid = lax.axis_index("c")
    sid = lax.axis_index("s")
    wid = sid * NCORES + cid
    base = wid * ROWS_PER_W

    pltpu.sync_copy(x_hbm.at[pl.ds(base, ROWS_PER_W)], idx_v)

    for b in range(NBUF - 1):
        pltpu.async_copy(table_hbm.at[idx_v.at[b]], bufs[b], sems[b])

    inv = jnp.full((LANES,), 1.0 / SEQ, dtype=jnp.float32)

    def block(buf, j, k):
        v = plsc.bitcast(buf[j, pl.ds(k * LANES, LANES)],
                         jnp.bfloat16)
        return plsc.unpack(v, format=plsc.PackFormat.INTERLEAVED)

    def accumulate(buf, i):
        acc = [block(buf, 0, k) for k in range(NBLK)]
        for j in range(1, SEQ):
            for k in range(NBLK):
                a, b = block(buf, j, k)
                bk = acc[k][1] + b if k < NBLK_HI else acc[k][1]
                acc[k] = (acc[k][0] + a, bk)
        for k in range(NBLK):
            out_v[2 * i, pl.ds(k * LANES, LANES)] = acc[k][0] * inv
        for k in range(NBLK_HI):
            out_v[2 * i + 1, pl.ds(k * LANES, LANES)] = acc[k][1] * inv

    def loop_body(j, carry):
        for b in range(NBUF):
            i = j + b
            nxt_row = jnp.minimum(i + NBUF - 1, ROWS_PER_W - 1)
            nb = (b + NBUF - 1) % NBUF
            pltpu.async_copy(table_hbm.at[idx_v.at[nxt_row]], bufs[nb],
                             sems[nb])
            pltpu.make_async_copy(table_hbm.at[idx_v.at[i]], bufs[b],
                                  sems[b]).wait()
            accumulate(bufs[b], i)
        return carry

    lax.fori_loop(0, ROWS_PER_W // NBUF,
                  lambda t, c: loop_body(t * NBUF, c), 0, unroll=False)

    for b in range(NBUF - 1):
        pltpu.make_async_copy(table_hbm.at[idx_v.at[0]], bufs[b],
                              sems[b]).wait()

    pltpu.sync_copy(out_v, out_hbm.at[pl.ds(2 * base, 2 * ROWS_PER_W)])


@functools.partial(
    pl.kernel,
    out_type=jax.ShapeDtypeStruct((2 * BATCH, 128), jnp.float32),
    mesh=plsc.VectorSubcoreMesh(core_axis_name="c", subcore_axis_name="s"),
    compiler_params=pltpu.CompilerParams(needs_layout_passes=False),
    scratch_types=[
        pltpu.VMEM((ROWS_PER_W, SEQ), jnp.int32),
        [pltpu.VMEM((SEQ, 128), jnp.int32)] * NBUF,
        pltpu.VMEM((2 * ROWS_PER_W, 128), jnp.float32),
        [pltpu.SemaphoreType.DMA] * NBUF,
    ],
)
def _pool(x_hbm, table_hbm, out_hbm, idx_v, bufs, out_v, sems):
    _pool_body(x_hbm, table_hbm, out_hbm, idx_v, bufs, out_v, sems)


RFMT_BLK = 10000


def _reformat_body(e_ref, o_ref):
    x = e_ref[...]
    y = jnp.pad(x, ((0, 0), (0, EMBED_P - EMBED)))
    y = y.astype(jnp.bfloat16)
    lo = jax.lax.bitcast_convert_type(y[:, :128], jnp.uint16)
    hi = jax.lax.bitcast_convert_type(y[:, 128:], jnp.uint16)
    o_ref[...] = (lo.astype(jnp.int32)
                  | (hi.astype(jnp.int32) << 16))


def _reformat(embed):
    return pl.pallas_call(
        _reformat_body,
        grid=(VOCAB // RFMT_BLK,),
        in_specs=[pl.BlockSpec((RFMT_BLK, EMBED), lambda i: (i, 0))],
        out_specs=pl.BlockSpec((RFMT_BLK, 128), lambda i: (i, 0)),
        out_shape=jax.ShapeDtypeStruct((VOCAB, 128), jnp.int32),
    )(embed)


MLP_BLK = 1024


def _mlp_body(p_ref, w1_ref, b1_ref, w2_ref, b2_ref, o_ref):
    h = jnp.dot(p_ref[...], w1_ref[...],
                preferred_element_type=jnp.float32) + b1_ref[...]
    h = jnp.maximum(h, 0.0)
    z = jnp.dot(h, w2_ref[...],
                preferred_element_type=jnp.float32) + b2_ref[...]
    m = jnp.max(z, axis=1, keepdims=True)
    lse = jnp.log(jnp.sum(jnp.exp(z - m), axis=1, keepdims=True)) + m
    o_ref[...] = z - lse


def _mlp(pooled, W1, b1, W2, b2):
    return pl.pallas_call(
        _mlp_body,
        grid=(BATCH // MLP_BLK,),
        in_specs=[
            pl.BlockSpec((MLP_BLK, EMBED), lambda i: (i, 0)),
            pl.BlockSpec((EMBED, HIDDEN), lambda i: (0, 0)),
            pl.BlockSpec((1, HIDDEN), lambda i: (0, 0)),
            pl.BlockSpec((HIDDEN, CLASSES), lambda i: (0, 0)),
            pl.BlockSpec((1, CLASSES), lambda i: (0, 0)),
        ],
        out_specs=pl.BlockSpec((MLP_BLK, CLASSES), lambda i: (i, 0)),
        out_shape=jax.ShapeDtypeStruct((BATCH, CLASSES), jnp.float32),
    )(pooled, W1, b1.reshape(1, HIDDEN), W2, b2.reshape(1, CLASSES))


def kernel(x, embed, W1, b1, W2, b2):
    xi = x.astype(jnp.int32)
    table = _reformat(embed)
    pooled2 = _pool(xi, table)
    pooled = pooled2.reshape(BATCH, EMBED_P)[:, :EMBED]
    return _mlp(pooled, W1, b1, W2, b2)


# speedup vs baseline: 2.3520x; 1.1014x over previous
import functools

import jax
import jax.numpy as jnp
from jax import lax
from jax.experimental import pallas as pl
from jax.experimental.pallas import tpu as pltpu
from jax.experimental.pallas import tpu_sc as plsc

VOCAB = 100000
EMBED = 200
EMBED_P = 256
HIDDEN = 64
CLASSES = 100
BATCH = 4096
SEQ = 50

LANES = 16
NCORES = 2
NSUB = 16
NWORKERS = NCORES * NSUB
ROWS_PER_W = BATCH // NWORKERS
NBLK = 128 // LANES
NBLK_HI = 5


NBUF = 3


def _pool_body(x_hbm, table_hbm, out_hbm, idx_v, bufs, out_v, sems):
    cid = lax.axis_index("c")
    sid = lax.axis_index("s")
    wid = sid * NCORES + cid
    base = wid * ROWS_PER_W

    pltpu.sync_copy(x_hbm.at[pl.ds(base, ROWS_PER_W)], idx_v)

    for b in range(NBUF - 1):
        pltpu.async_copy(table_hbm.at[idx_v.at[b]], bufs[b], sems[b])

    inv = jnp.full((LANES,), 1.0 / SEQ, dtype=jnp.float32)

    def block(buf, j, k):
        v = plsc.bitcast(buf[j, pl.ds(k * LANES, LANES)],
                         jnp.bfloat16)
        return plsc.unpack(v, format=plsc.PackFormat.INTERLEAVED)

    def accumulate(buf, i):
        acc = [block(buf, 0, k) for k in range(NBLK)]
        for j in range(1, SEQ):
            for k in range(NBLK):
                a, b = block(buf, j, k)
                bk = acc[k][1] + b if k < NBLK_HI else acc[k][1]
                acc[k] = (acc[k][0] + a, bk)
        for k in range(NBLK):
            out_v[2 * i, pl.ds(k * LANES, LANES)] = acc[k][0] * inv
        for k in range(NBLK_HI):
            out_v[2 * i + 1, pl.ds(k * LANES, LANES)] = acc[k][1] * inv

    def loop_body(j, carry):
        for b in range(NBUF):
            i = j + b
            nxt_row = jnp.minimum(i + NBUF - 1, ROWS_PER_W - 1)
            nb = (b + NBUF - 1) % NBUF
            pltpu.async_copy(table_hbm.at[idx_v.at[nxt_row]], bufs[nb],
                             sems[nb])
            pltpu.make_async_copy(table_hbm.at[idx_v.at[i]], bufs[b],
                                  sems[b]).wait()
            accumulate(bufs[b], i)
        return carry

    lax.fori_loop(0, ROWS_PER_W // NBUF,
                  lambda t, c: loop_body(t * NBUF, c), 0, unroll=False)

    for b in range(NBUF - 1):
        pltpu.make_async_copy(table_hbm.at[idx_v.at[0]], bufs[b],
                              sems[b]).wait()

    pltpu.sync_copy(out_v, out_hbm.at[pl.ds(2 * base, 2 * ROWS_PER_W)])


@functools.partial(
    pl.kernel,
    out_type=jax.ShapeDtypeStruct((2 * BATCH, 128), jnp.float32),
    mesh=plsc.VectorSubcoreMesh(core_axis_name="c", subcore_axis_name="s"),
    compiler_params=pltpu.CompilerParams(needs_layout_passes=False),
    scratch_types=[
        pltpu.VMEM((ROWS_PER_W, SEQ), jnp.int32),
        [pltpu.VMEM((SEQ, 128), jnp.int32)] * NBUF,
        pltpu.VMEM((2 * ROWS_PER_W, 128), jnp.float32),
        [pltpu.SemaphoreType.DMA] * NBUF,
    ],
)
def _pool(x_hbm, table_hbm, out_hbm, idx_v, bufs, out_v, sems):
    _pool_body(x_hbm, table_hbm, out_hbm, idx_v, bufs, out_v, sems)


RFMT_BLK = 10000


def _reformat_body(e_ref, o_ref):
    x = e_ref[...]
    y = jnp.pad(x, ((0, 0), (0, EMBED_P - EMBED)))
    y = y.astype(jnp.bfloat16)
    lo = jax.lax.bitcast_convert_type(y[:, :128], jnp.uint16)
    hi = jax.lax.bitcast_convert_type(y[:, 128:], jnp.uint16)
    o_ref[...] = (lo.astype(jnp.int32)
                  | (hi.astype(jnp.int32) << 16))


def _reformat(embed):
    return pl.pallas_call(
        _reformat_body,
        grid=(VOCAB // RFMT_BLK,),
        in_specs=[pl.BlockSpec((RFMT_BLK, EMBED), lambda i: (i, 0))],
        out_specs=pl.BlockSpec((RFMT_BLK, 128), lambda i: (i, 0)),
        out_shape=jax.ShapeDtypeStruct((VOCAB, 128), jnp.int32),
    )(embed)


MLP_BLK = 1024


def _mlp_body(p_ref, w1_ref, b1_ref, w2_ref, b2_ref, o_ref):
    h = jnp.dot(p_ref[...], w1_ref[...],
                preferred_element_type=jnp.float32) + b1_ref[...]
    h = jnp.maximum(h, 0.0)
    z = jnp.dot(h, w2_ref[...],
                preferred_element_type=jnp.float32) + b2_ref[...]
    m = jnp.max(z, axis=1, keepdims=True)
    lse = jnp.log(jnp.sum(jnp.exp(z - m), axis=1, keepdims=True)) + m
    o_ref[...] = z - lse


def _mlp(pooled, W1, b1, W2, b2):
    return pl.pallas_call(
        _mlp_body,
        grid=(BATCH // MLP_BLK,),
        in_specs=[
            pl.BlockSpec((MLP_BLK, EMBED), lambda i: (i, 0)),
            pl.BlockSpec((EMBED, HIDDEN), lambda i: (0, 0)),
            pl.BlockSpec((1, HIDDEN), lambda i: (0, 0)),
            pl.BlockSpec((HIDDEN, CLASSES), lambda i: (0, 0)),
            pl.BlockSpec((1, CLASSES), lambda i: (0, 0)),
        ],
        out_specs=pl.BlockSpec((MLP_BLK, CLASSES), lambda i: (i, 0)),
        out_shape=jax.ShapeDtypeStruct((BATCH, CLASSES), jnp.float32),
    )(pooled, W1, b1.reshape(1, HIDDEN), W2, b2.reshape(1, CLASSES))


def kernel(x, embed, W1, b1, W2, b2):
    xi = x.astype(jnp.int32)
    table = _reformat(embed)
    pooled2 = _pool(xi, table)
    pooled = pooled2.reshape(BATCH, EMBED_P)[:, :EMBED]
    return _mlp(pooled, W1, b1, W2, b2)
